# baseline (speedup 1.0000x reference)
import functools

import jax
import jax.numpy as jnp
from jax import lax
from jax.experimental import pallas as pl
from jax.experimental.pallas import tpu as pltpu
from jax.experimental.pallas import tpu_sc as plsc

STEPS = 3
N = 10000
D = 128
NG = 16
E = 320000

NC = 2
NS = 16
L = 16
NW = NC * NS

NPT = 320
NPAD = NW * NPT
CE = 4000
NCHUNK = E // CE
GRP = 128
NEG = float("-inf")


def _leaky(v):
    return jnp.where(v >= 0, v, 0.01 * v)


def _dot_t(a, b):
    return lax.dot_general(a, b, (((1,), (1,)), ((), ())),
                           preferred_element_type=jnp.float32)


def _mess_body(x_ref, w_ref, b_ref, y_ref):
    y_ref[...] = _dot_t(x_ref[...], w_ref[...]) + b_ref[...]


def _mess_call(x, w, b):
    return pl.pallas_call(
        _mess_body,
        out_shape=jax.ShapeDtypeStruct((N, D), jnp.float32),
    )(x, w, b)


def _step_body(x_ref, xg_ref, araw_ref, oh_ref, faw_ref, fab_ref,
               amw16_ref, amb_ref, afw_ref, afb_ref, trw_ref, trb_ref,
               wmn_ref, bmn_ref, xn_ref, xgn_ref, yn_ref):
    x = x_ref[...]
    xg = xg_ref[...]
    araw = araw_ref[...]
    aggr = jnp.where(jnp.isfinite(araw), _leaky(araw), 0.0)

    faw = faw_ref[...]
    wx = faw[:, :D]
    wg = faw[:, D:2 * D]
    wa = faw[:, 2 * D:]

    oh = oh_ref[...]

    g16 = _dot_t(xg, wg)
    t = (_dot_t(x, wx)
         + jnp.dot(oh, g16, preferred_element_type=jnp.float32)
         + _dot_t(aggr, wa)
         + fab_ref[...])
    xn = _leaky(t) + x

    gate16 = _dot_t(xn, amw16_ref[...]) + amb_ref[0, 0]
    masked = jnp.where(oh > 0, gate16, NEG)
    gmax = jnp.max(masked, axis=0, keepdims=True)
    gmax = jnp.where(jnp.isfinite(gmax), gmax, 0.0)
    exm = jnp.exp(jnp.where(oh > 0, gate16 - gmax, NEG))
    den = jnp.sum(exm, axis=0, keepdims=True)
    alpha16 = exm / (den + 1e-16)
    feat = _leaky(_dot_t(xn, afw_ref[...]) + afb_ref[...])
    pooled = lax.dot_general(alpha16, feat, (((0,), (0,)), ((), ())),
                             preferred_element_type=jnp.float32)

    trw = trw_ref[...]
    xgn = _leaky(_dot_t(pooled, trw[:, :D]) + _dot_t(xg, trw[:, D:])
                 + trb_ref[...]) + xg

    xn_ref[...] = xn
    xgn_ref[...] = xgn
    yn_ref[...] = _dot_t(xn, wmn_ref[...]) + bmn_ref[...]


def _step_call(x, xg, araw, oh, faw, fab, amw16, amb, afw, afb, trw, trb,
               wmn, bmn):
    return pl.pallas_call(
        _step_body,
        out_shape=(
            jax.ShapeDtypeStruct((N, D), jnp.float32),
            jax.ShapeDtypeStruct((NG, D), jnp.float32),
            jax.ShapeDtypeStruct((N, D), jnp.float32),
        ),
    )(x, xg, araw, oh, faw, fab, amw16, amb, afw, afb, trw, trb, wmn, bmn)


def _segmax_body(y_hbm, src_hbm, dst_hbm, out_hbm,
                 acc, dstb, srcb, cdst, csrc, idxg, rows, sem):
    wid = lax.axis_index("s") * NC + lax.axis_index("c")
    lo = wid * NPT
    hi = lo + NPT

    def init_row(r, _):
        for f in range(D // L):
            acc[r, pl.ds(f * L, L)] = jnp.full((L,), NEG, jnp.float32)
        return 0
    lax.fori_loop(0, NPT, init_row, 0)

    zeros16 = jnp.zeros((L,), jnp.int32)

    def chunk_body(c, _):
        pltpu.sync_copy(dst_hbm.at[pl.ds(c * CE, CE)], dstb)
        pltpu.sync_copy(src_hbm.at[pl.ds(c * CE, CE)], srcb)

        def scan_body(k, p):
            d16 = dstb[pl.ds(k * L, L)]
            s16 = srcb[pl.ds(k * L, L)]
            m = (d16 >= lo) & (d16 < hi)
            mi = m.astype(jnp.int32)
            pos = p + plsc.cumsum(mi) - mi
            plsc.store_scatter(cdst, [pos], d16, mask=m)
            plsc.store_scatter(csrc, [pos], s16, mask=m)
            return p + jnp.sum(mi)

        na = lax.fori_loop(0, CE // L, scan_body, 0)

        for t in range(GRP // L):
            csrc[pl.ds(na + t * L, L)] = zeros16

        def grp_body(gi, _):
            base = gi * GRP
            g = jnp.minimum(na - base, GRP)
            for t in range(GRP // L):
                idxg[pl.ds(t * L, L)] = csrc[pl.ds(base + t * L, L)]
            pltpu.async_copy(y_hbm.at[idxg], rows, sem).wait()

            def edge_body(j, _):
                dl = cdst[pl.ds(base + j, L)][0] - lo
                for f in range(D // L):
                    sl = pl.ds(f * L, L)
                    acc[dl, sl] = jnp.maximum(acc[dl, sl], rows[j, sl])
                return 0

            lax.fori_loop(0, g, edge_body, 0)
            return 0

        ngrp = (na + GRP - 1) // GRP
        lax.fori_loop(0, ngrp, grp_body, 0)
        return 0

    lax.fori_loop(0, NCHUNK, chunk_body, 0)
    pltpu.sync_copy(acc, out_hbm.at[pl.ds(lo, NPT)])


_segmax_call = pl.kernel(
    _segmax_body,
    out_type=jax.ShapeDtypeStruct((NPAD, D), jnp.float32),
    mesh=plsc.VectorSubcoreMesh(core_axis_name="c", subcore_axis_name="s"),
    compiler_params=pltpu.CompilerParams(needs_layout_passes=False),
    scratch_types=[
        pltpu.VMEM((NPT, D), jnp.float32),
        pltpu.VMEM((CE,), jnp.int32),
        pltpu.VMEM((CE,), jnp.int32),
        pltpu.VMEM((CE + GRP + L,), jnp.int32),
        pltpu.VMEM((CE + GRP + L,), jnp.int32),
        pltpu.VMEM((GRP,), jnp.int32),
        pltpu.VMEM((GRP, D), jnp.float32),
        pltpu.SemaphoreType.DMA,
    ],
)


def kernel(x, x_global, edge_attr, edge_index, batch_ind, num_graphs,
           f_mess_W, f_mess_b, f_agg_W, f_agg_b, att_mask_W, att_mask_b,
           att_feat_W, att_feat_b, transform_W, transform_b):
    del edge_attr, num_graphs
    src = edge_index[0]
    dst = edge_index[1]
    oh = (batch_ind[:, None] == jnp.arange(NG, dtype=batch_ind.dtype)[None, :]
          ).astype(jnp.float32)

    y = _mess_call(x, f_mess_W[0], f_mess_b[0].reshape(1, D))
    for i in range(STEPS):
        araw = _segmax_call(y, src, dst)[:N]
        nxt = (i + 1) % STEPS
        x, x_global, y = _step_call(
            x, x_global, araw, oh,
            f_agg_W[i], f_agg_b[i].reshape(1, D),
            jnp.tile(att_mask_W[i], (NG, 1)), att_mask_b[i].reshape(1, 1),
            att_feat_W[i], att_feat_b[i].reshape(1, D),
            transform_W[i], transform_b[i].reshape(1, D),
            f_mess_W[nxt], f_mess_b[nxt].reshape(1, D),
        )
    return (x, x_global)


# speedup vs baseline: 4.3319x; 4.3319x over previous
import functools

import jax
import jax.numpy as jnp
from jax import lax
from jax.experimental import pallas as pl
from jax.experimental.pallas import tpu as pltpu
from jax.experimental.pallas import tpu_sc as plsc

STEPS = 3
N = 10000
D = 128
NG = 16
E = 320000

NC = 2
NS = 16
L = 16
NW = NC * NS

NPT = 320
NPAD = NW * NPT
CE = 4000
NCHUNK = E // CE
GRP = 128
NEG = float("-inf")


def _leaky(v):
    return jnp.where(v >= 0, v, 0.01 * v)


def _dot_t(a, b):
    return lax.dot_general(a, b, (((1,), (1,)), ((), ())),
                           preferred_element_type=jnp.float32)


def _mess_body(x_ref, w_ref, b_ref, y_ref):
    y_ref[...] = _dot_t(x_ref[...], w_ref[...]) + b_ref[...]


def _mess_call(x, w, b):
    return pl.pallas_call(
        _mess_body,
        out_shape=jax.ShapeDtypeStruct((N, D), jnp.float32),
    )(x, w, b)


def _step_body(x_ref, xg_ref, araw_ref, oh_ref, faw_ref, fab_ref,
               amw16_ref, amb_ref, afw_ref, afb_ref, trw_ref, trb_ref,
               wmn_ref, bmn_ref, xn_ref, xgn_ref, yn_ref):
    x = x_ref[...]
    xg = xg_ref[...]
    araw = araw_ref[...]
    aggr = jnp.where(jnp.isfinite(araw), _leaky(araw), 0.0)

    faw = faw_ref[...]
    wx = faw[:, :D]
    wg = faw[:, D:2 * D]
    wa = faw[:, 2 * D:]

    oh = oh_ref[...]

    g16 = _dot_t(xg, wg)
    t = (_dot_t(x, wx)
         + jnp.dot(oh, g16, preferred_element_type=jnp.float32)
         + _dot_t(aggr, wa)
         + fab_ref[...])
    xn = _leaky(t) + x

    gate16 = _dot_t(xn, amw16_ref[...]) + amb_ref[0, 0]
    masked = jnp.where(oh > 0, gate16, NEG)
    gmax = jnp.max(masked, axis=0, keepdims=True)
    gmax = jnp.where(jnp.isfinite(gmax), gmax, 0.0)
    exm = jnp.exp(jnp.where(oh > 0, gate16 - gmax, NEG))
    den = jnp.sum(exm, axis=0, keepdims=True)
    alpha16 = exm / (den + 1e-16)
    feat = _leaky(_dot_t(xn, afw_ref[...]) + afb_ref[...])
    pooled = lax.dot_general(alpha16, feat, (((0,), (0,)), ((), ())),
                             preferred_element_type=jnp.float32)

    trw = trw_ref[...]
    xgn = _leaky(_dot_t(pooled, trw[:, :D]) + _dot_t(xg, trw[:, D:])
                 + trb_ref[...]) + xg

    xn_ref[...] = xn
    xgn_ref[...] = xgn
    yn_ref[...] = _dot_t(xn, wmn_ref[...]) + bmn_ref[...]


def _step_call(x, xg, araw, oh, faw, fab, amw16, amb, afw, afb, trw, trb,
               wmn, bmn):
    return pl.pallas_call(
        _step_body,
        out_shape=(
            jax.ShapeDtypeStruct((N, D), jnp.float32),
            jax.ShapeDtypeStruct((NG, D), jnp.float32),
            jax.ShapeDtypeStruct((N, D), jnp.float32),
        ),
    )(x, xg, araw, oh, faw, fab, amw16, amb, afw, afb, trw, trb, wmn, bmn)


CE2 = E // NW
REGCAP = CE2 + NW * L
EALLOC = NW * REGCAP + 1024
TABC = 96
SENT = 1 << 30


def _part_body(src_hbm, dst_hbm, srcp_hbm, dstp_hbm, tab_hbm,
               srcc, dstc, srcl, dstl, bins, cur, tabl):
    wid = lax.axis_index("s") * NC + lax.axis_index("c")
    base = wid * CE2
    base = pl.multiple_of(base, 16)
    pltpu.sync_copy(src_hbm.at[pl.ds(base, CE2)], srcc)
    pltpu.sync_copy(dst_hbm.at[pl.ds(base, CE2)], dstc)

    lane = lax.iota(jnp.int32, L)
    zero16 = jnp.zeros((L,), jnp.int32)
    one16 = jnp.ones((L,), jnp.int32)
    for i in range(NW * L // L):
        bins[pl.ds(i * L, L)] = zero16

    def hist(k, _):
        d16 = dstc[pl.ds(k * L, L)]
        b16 = (d16 * 6554) >> 21
        plsc.addupdate_scatter(bins, [lane * NW + b16], one16)
        return 0
    lax.fori_loop(0, CE2 // L, hist, 0)

    c0 = zero16
    c1 = zero16
    for l in range(L):
        c0 = c0 + bins[pl.ds(l * NW, L)]
        c1 = c1 + bins[pl.ds(l * NW + L, L)]
    p0 = ((c0 + 15) >> 4) << 4
    p1 = ((c1 + 15) >> 4) << 4
    s0 = plsc.cumsum(p0)
    s1 = plsc.cumsum(p1) + s0[L - 1]
    st0 = s0 - p0
    st1 = s1 - p1

    for i in range(TABC // L):
        tabl[pl.ds(i * L, L)] = zero16
    tabl[pl.ds(0, L)] = st0
    tabl[pl.ds(L, L)] = st1
    tabl[pl.ds(2 * L, L)] = c0
    tabl[pl.ds(3 * L, L)] = c1
    cur[pl.ds(0, L)] = st0
    cur[pl.ds(L, L)] = st1

    sent16 = jnp.full((L,), SENT, jnp.int32)

    def fill(r, _):
        srcl[pl.ds(r * L, L)] = zero16
        dstl[pl.ds(r * L, L)] = sent16
        return 0
    lax.fori_loop(0, REGCAP // L, fill, 0)

    mask0 = lane < 1
    inc0 = (lane == 0).astype(jnp.int32)

    def rank16(k, _):
        d16 = dstc[pl.ds(k * L, L)]
        s16 = srcc[pl.ds(k * L, L)]
        b16 = (d16 * 6554) >> 21
        for l in range(L):
            b = b16[l]
            cur16 = cur[pl.ds(b, L)]
            pos = cur16[0]
            cur[pl.ds(b, L)] = cur16 + inc0
            posv = zero16 + pos
            plsc.store_scatter(srcl, [posv], zero16 + s16[l], mask=mask0)
            plsc.store_scatter(dstl, [posv], zero16 + d16[l], mask=mask0)
        return 0
    lax.fori_loop(0, CE2 // L, rank16, 0)

    reg0 = pl.multiple_of(wid * REGCAP, 16)
    pltpu.sync_copy(srcl, srcp_hbm.at[pl.ds(reg0, REGCAP)])
    pltpu.sync_copy(dstl, dstp_hbm.at[pl.ds(reg0, REGCAP)])

    @pl.when(wid == NW - 1)
    def _():
        def tail_fill(r, _):
            srcl[pl.ds(r * L, L)] = zero16
            dstl[pl.ds(r * L, L)] = sent16
            return 0
        lax.fori_loop(0, 1024 // L, tail_fill, 0)
        pltpu.sync_copy(srcl.at[pl.ds(0, 1024)],
                        srcp_hbm.at[pl.ds(NW * REGCAP, 1024)])
        pltpu.sync_copy(dstl.at[pl.ds(0, 1024)],
                        dstp_hbm.at[pl.ds(NW * REGCAP, 1024)])
    pltpu.sync_copy(tabl, tab_hbm.at[pl.ds(pl.multiple_of(wid * TABC, 16), TABC)])


_part_call = pl.kernel(
    _part_body,
    out_type=(
        jax.ShapeDtypeStruct((EALLOC,), jnp.int32),
        jax.ShapeDtypeStruct((EALLOC,), jnp.int32),
        jax.ShapeDtypeStruct((NW * TABC,), jnp.int32),
    ),
    mesh=plsc.VectorSubcoreMesh(core_axis_name="c", subcore_axis_name="s"),
    compiler_params=pltpu.CompilerParams(needs_layout_passes=False),
    scratch_types=[
        pltpu.VMEM((CE2,), jnp.int32),
        pltpu.VMEM((CE2,), jnp.int32),
        pltpu.VMEM((REGCAP,), jnp.int32),
        pltpu.VMEM((REGCAP,), jnp.int32),
        pltpu.VMEM((NW * L,), jnp.int32),
        pltpu.VMEM((NW + L,), jnp.int32),
        pltpu.VMEM((TABC,), jnp.int32),
    ],
)


BLK = 512


def _segmax_body(y_hbm, srcp_hbm, dstp_hbm, tab_hbm, out_hbm,
                 acc, trow, sblk, dblk, idxg, rows, sem):
    wid = lax.axis_index("s") * NC + lax.axis_index("c")
    lo = wid * NPT
    hi = lo + NPT

    neg16 = jnp.full((L,), NEG, jnp.float32)

    def init_row(r, _):
        for f in range(D // L):
            acc[r, pl.ds(f * L, L)] = neg16
        return 0
    lax.fori_loop(0, NPT + 1, init_row, 0)

    def prod_body(t, _):
        pltpu.sync_copy(tab_hbm.at[pl.ds(pl.multiple_of(t * TABC, 16), TABC)], trow)
        st = trow[pl.ds(wid, L)][0]
        cnt = trow[pl.ds(2 * L + wid, L)][0]
        abs0 = t * REGCAP + st
        pc = ((cnt + 15) >> 4) << 4
        nblk = (pc + BLK - 1) >> 9

        def blk_body(b, _):
            off = pl.multiple_of(abs0 + b * BLK, 16)
            pltpu.sync_copy(srcp_hbm.at[pl.ds(off, BLK)], sblk)
            pltpu.sync_copy(dstp_hbm.at[pl.ds(off, BLK)], dblk)
            for g in range(BLK // GRP):
                for tt in range(GRP // L):
                    v = sblk[pl.ds(g * GRP + tt * L, L)]
                    idxg[pl.ds(tt * L, L)] = jnp.minimum(
                        jnp.maximum(v, 0), N - 1)
                pltpu.async_copy(y_hbm.at[idxg], rows, sem).wait()

                def rmw(jj, _):
                    d16 = dblk[pl.ds(g * GRP + jj * L, L)]
                    for l in range(L):
                        d = d16[l]
                        inr = (d >= lo) & (d < hi)
                        dl = jnp.where(inr, d - lo, NPT)
                        j = jj * L + l
                        for f in range(D // L):
                            sl = pl.ds(f * L, L)
                            acc[dl, sl] = jnp.maximum(acc[dl, sl], rows[j, sl])
                    return 0
                lax.fori_loop(0, GRP // L, rmw, 0)
            return 0

        lax.fori_loop(0, nblk, blk_body, 0)
        return 0

    lax.fori_loop(0, NW, prod_body, 0)
    pltpu.sync_copy(acc.at[pl.ds(0, NPT)], out_hbm.at[pl.ds(lo, NPT)])


_segmax_call = pl.kernel(
    _segmax_body,
    out_type=jax.ShapeDtypeStruct((NPAD, D), jnp.float32),
    mesh=plsc.VectorSubcoreMesh(core_axis_name="c", subcore_axis_name="s"),
    compiler_params=pltpu.CompilerParams(needs_layout_passes=False),
    scratch_types=[
        pltpu.VMEM((NPT + 1, D), jnp.float32),
        pltpu.VMEM((TABC,), jnp.int32),
        pltpu.VMEM((BLK,), jnp.int32),
        pltpu.VMEM((BLK,), jnp.int32),
        pltpu.VMEM((GRP,), jnp.int32),
        pltpu.VMEM((GRP, D), jnp.float32),
        pltpu.SemaphoreType.DMA,
    ],
)


def kernel(x, x_global, edge_attr, edge_index, batch_ind, num_graphs,
           f_mess_W, f_mess_b, f_agg_W, f_agg_b, att_mask_W, att_mask_b,
           att_feat_W, att_feat_b, transform_W, transform_b):
    del edge_attr, num_graphs
    src = edge_index[0]
    dst = edge_index[1]
    oh = (batch_ind[:, None] == jnp.arange(NG, dtype=batch_ind.dtype)[None, :]
          ).astype(jnp.float32)

    srcp, dstp, tab = _part_call(src, dst)
    y = _mess_call(x, f_mess_W[0], f_mess_b[0].reshape(1, D))
    for i in range(STEPS):
        araw = _segmax_call(y, srcp, dstp, tab)[:N]
        nxt = (i + 1) % STEPS
        x, x_global, y = _step_call(
            x, x_global, araw, oh,
            f_agg_W[i], f_agg_b[i].reshape(1, D),
            jnp.tile(att_mask_W[i], (NG, 1)), att_mask_b[i].reshape(1, 1),
            att_feat_W[i], att_feat_b[i].reshape(1, D),
            transform_W[i], transform_b[i].reshape(1, D),
            f_mess_W[nxt], f_mess_b[nxt].reshape(1, D),
        )
    return (x, x_global)


# speedup vs baseline: 5.0124x; 1.1571x over previous
import functools

import jax
import jax.numpy as jnp
from jax import lax
from jax.experimental import pallas as pl
from jax.experimental.pallas import tpu as pltpu
from jax.experimental.pallas import tpu_sc as plsc

STEPS = 3
N = 10000
D = 128
NG = 16
E = 320000

NC = 2
NS = 16
L = 16
NW = NC * NS

NPT = 320
NPAD = NW * NPT
CE = 4000
NCHUNK = E // CE
GRP = 128
NEG = float("-inf")


def _leaky(v):
    return jnp.where(v >= 0, v, 0.01 * v)


def _dot_t(a, b):
    return lax.dot_general(a, b, (((1,), (1,)), ((), ())),
                           preferred_element_type=jnp.float32)


def _mess_body(x_ref, w_ref, b_ref, y_ref):
    y_ref[...] = _dot_t(x_ref[...], w_ref[...]) + b_ref[...]


def _mess_call(x, w, b):
    return pl.pallas_call(
        _mess_body,
        out_shape=jax.ShapeDtypeStruct((N, D), jnp.float32),
    )(x, w, b)


def _step_body(x_ref, xg_ref, araw_ref, oh_ref, faw_ref, fab_ref,
               amw16_ref, amb_ref, afw_ref, afb_ref, trw_ref, trb_ref,
               wmn_ref, bmn_ref, xn_ref, xgn_ref, yn_ref):
    x = x_ref[...]
    xg = xg_ref[...]
    araw = araw_ref[...]
    aggr = jnp.where(jnp.isfinite(araw), _leaky(araw), 0.0)

    faw = faw_ref[...]
    wx = faw[:, :D]
    wg = faw[:, D:2 * D]
    wa = faw[:, 2 * D:]

    oh = oh_ref[...]

    g16 = _dot_t(xg, wg)
    t = (_dot_t(x, wx)
         + jnp.dot(oh, g16, preferred_element_type=jnp.float32)
         + _dot_t(aggr, wa)
         + fab_ref[...])
    xn = _leaky(t) + x

    gate16 = _dot_t(xn, amw16_ref[...]) + amb_ref[0, 0]
    masked = jnp.where(oh > 0, gate16, NEG)
    gmax = jnp.max(masked, axis=0, keepdims=True)
    gmax = jnp.where(jnp.isfinite(gmax), gmax, 0.0)
    exm = jnp.exp(jnp.where(oh > 0, gate16 - gmax, NEG))
    den = jnp.sum(exm, axis=0, keepdims=True)
    alpha16 = exm / (den + 1e-16)
    feat = _leaky(_dot_t(xn, afw_ref[...]) + afb_ref[...])
    pooled = lax.dot_general(alpha16, feat, (((0,), (0,)), ((), ())),
                             preferred_element_type=jnp.float32)

    trw = trw_ref[...]
    xgn = _leaky(_dot_t(pooled, trw[:, :D]) + _dot_t(xg, trw[:, D:])
                 + trb_ref[...]) + xg

    xn_ref[...] = xn
    xgn_ref[...] = xgn
    yn_ref[...] = _dot_t(xn, wmn_ref[...]) + bmn_ref[...]


def _step_call(x, xg, araw, oh, faw, fab, amw16, amb, afw, afb, trw, trb,
               wmn, bmn):
    return pl.pallas_call(
        _step_body,
        out_shape=(
            jax.ShapeDtypeStruct((N, D), jnp.float32),
            jax.ShapeDtypeStruct((NG, D), jnp.float32),
            jax.ShapeDtypeStruct((N, D), jnp.float32),
        ),
    )(x, xg, araw, oh, faw, fab, amw16, amb, afw, afb, trw, trb, wmn, bmn)


CE2 = E // NW
REGCAP = CE2 + NW * L
EALLOC = NW * REGCAP + 1024
TABC = 96
SENT = 1 << 30


def _part_body(src_hbm, dst_hbm, srcp_hbm, dstp_hbm, tab_hbm,
               srcc, dstc, srcl, dstl, bins, cur, tabl):
    wid = lax.axis_index("s") * NC + lax.axis_index("c")
    base = wid * CE2
    base = pl.multiple_of(base, 16)
    pltpu.sync_copy(src_hbm.at[pl.ds(base, CE2)], srcc)
    pltpu.sync_copy(dst_hbm.at[pl.ds(base, CE2)], dstc)

    lane = lax.iota(jnp.int32, L)
    zero16 = jnp.zeros((L,), jnp.int32)
    one16 = jnp.ones((L,), jnp.int32)
    for i in range(NW * L // L):
        bins[pl.ds(i * L, L)] = zero16

    def hist(k, _):
        d16 = dstc[pl.ds(k * L, L)]
        b16 = (d16 * 6554) >> 21
        plsc.addupdate_scatter(bins, [lane * NW + b16], one16)
        return 0
    lax.fori_loop(0, CE2 // L, hist, 0)

    c0 = zero16
    c1 = zero16
    for l in range(L):
        c0 = c0 + bins[pl.ds(l * NW, L)]
        c1 = c1 + bins[pl.ds(l * NW + L, L)]
    p0 = ((c0 + 15) >> 4) << 4
    p1 = ((c1 + 15) >> 4) << 4
    s0 = plsc.cumsum(p0)
    s1 = plsc.cumsum(p1) + s0[L - 1]
    st0 = s0 - p0
    st1 = s1 - p1

    for i in range(TABC // L):
        tabl[pl.ds(i * L, L)] = zero16
    tabl[pl.ds(0, L)] = st0
    tabl[pl.ds(L, L)] = st1
    tabl[pl.ds(2 * L, L)] = c0
    tabl[pl.ds(3 * L, L)] = c1
    cur[pl.ds(0, L)] = st0
    cur[pl.ds(L, L)] = st1

    sent16 = jnp.full((L,), SENT, jnp.int32)

    def fill(r, _):
        srcl[pl.ds(r * L, L)] = zero16
        dstl[pl.ds(r * L, L)] = sent16
        return 0
    lax.fori_loop(0, REGCAP // L, fill, 0)

    mask0 = lane < 1
    inc0 = (lane == 0).astype(jnp.int32)

    def rank16(k, _):
        d16 = dstc[pl.ds(k * L, L)]
        s16 = srcc[pl.ds(k * L, L)]
        b16 = (d16 * 6554) >> 21
        for l in range(L):
            b = b16[l]
            cur16 = cur[pl.ds(b, L)]
            pos = cur16[0]
            cur[pl.ds(b, L)] = cur16 + inc0
            posv = zero16 + pos
            plsc.store_scatter(srcl, [posv], zero16 + s16[l], mask=mask0)
            plsc.store_scatter(dstl, [posv], zero16 + d16[l], mask=mask0)
        return 0
    lax.fori_loop(0, CE2 // L, rank16, 0)

    reg0 = pl.multiple_of(wid * REGCAP, 16)
    pltpu.sync_copy(srcl, srcp_hbm.at[pl.ds(reg0, REGCAP)])
    pltpu.sync_copy(dstl, dstp_hbm.at[pl.ds(reg0, REGCAP)])

    @pl.when(wid == NW - 1)
    def _():
        def tail_fill(r, _):
            srcl[pl.ds(r * L, L)] = zero16
            dstl[pl.ds(r * L, L)] = sent16
            return 0
        lax.fori_loop(0, 1024 // L, tail_fill, 0)
        pltpu.sync_copy(srcl.at[pl.ds(0, 1024)],
                        srcp_hbm.at[pl.ds(NW * REGCAP, 1024)])
        pltpu.sync_copy(dstl.at[pl.ds(0, 1024)],
                        dstp_hbm.at[pl.ds(NW * REGCAP, 1024)])
    pltpu.sync_copy(tabl, tab_hbm.at[pl.ds(pl.multiple_of(wid * TABC, 16), TABC)])


_part_call = pl.kernel(
    _part_body,
    out_type=(
        jax.ShapeDtypeStruct((EALLOC,), jnp.int32),
        jax.ShapeDtypeStruct((EALLOC,), jnp.int32),
        jax.ShapeDtypeStruct((NW * TABC,), jnp.int32),
    ),
    mesh=plsc.VectorSubcoreMesh(core_axis_name="c", subcore_axis_name="s"),
    compiler_params=pltpu.CompilerParams(needs_layout_passes=False),
    scratch_types=[
        pltpu.VMEM((CE2,), jnp.int32),
        pltpu.VMEM((CE2,), jnp.int32),
        pltpu.VMEM((REGCAP,), jnp.int32),
        pltpu.VMEM((REGCAP,), jnp.int32),
        pltpu.VMEM((NW * L,), jnp.int32),
        pltpu.VMEM((NW + L,), jnp.int32),
        pltpu.VMEM((TABC,), jnp.int32),
    ],
)


BLK = 512


def _segmax_body(y_hbm, srcp_hbm, dstp_hbm, tab_hbm, out_hbm,
                 acc, trow, sblk, dblk, idxg, rows, sem):
    wid = lax.axis_index("s") * NC + lax.axis_index("c")
    lo = wid * NPT
    hi = lo + NPT

    neg16 = jnp.full((L,), NEG, jnp.float32)

    def init_row(r, _):
        for f in range(D // L):
            acc[r, pl.ds(f * L, L)] = neg16
        return 0
    lax.fori_loop(0, NPT + 1, init_row, 0)

    def prod_body(t, _):
        pltpu.sync_copy(tab_hbm.at[pl.ds(pl.multiple_of(t * TABC, 16), TABC)], trow)
        st = trow[pl.ds(wid, L)][0]
        cnt = trow[pl.ds(2 * L + wid, L)][0]
        abs0 = t * REGCAP + st
        pc = ((cnt + 15) >> 4) << 4
        nblk = (pc + BLK - 1) >> 9

        def blk_body(b, _):
            off = pl.multiple_of(abs0 + b * BLK, 16)
            pltpu.sync_copy(srcp_hbm.at[pl.ds(off, BLK)], sblk)
            pltpu.sync_copy(dstp_hbm.at[pl.ds(off, BLK)], dblk)
            for g in range(BLK // GRP):
                for tt in range(GRP // L):
                    v = sblk[pl.ds(g * GRP + tt * L, L)]
                    idxg[pl.ds(tt * L, L)] = jnp.minimum(
                        jnp.maximum(v, 0), N - 1)
                pltpu.async_copy(y_hbm.at[idxg], rows, sem).wait()

                def rmw(jj, _):
                    d16 = dblk[pl.ds(g * GRP + jj * L, L)]
                    for l in range(L):
                        d = d16[l]
                        inr = (d >= lo) & (d < hi)
                        dl = jnp.where(inr, d - lo, NPT)
                        j = jj * L + l
                        av = [acc[dl, pl.ds(f * L, L)] for f in range(D // L)]
                        rv = [rows[j, pl.ds(f * L, L)] for f in range(D // L)]
                        mv = [jnp.maximum(a, r) for a, r in zip(av, rv)]
                        for f in range(D // L):
                            acc[dl, pl.ds(f * L, L)] = mv[f]
                    return 0
                lax.fori_loop(0, GRP // L, rmw, 0)
            return 0

        lax.fori_loop(0, nblk, blk_body, 0)
        return 0

    lax.fori_loop(0, NW, prod_body, 0)
    pltpu.sync_copy(acc.at[pl.ds(0, NPT)], out_hbm.at[pl.ds(lo, NPT)])


_segmax_call = pl.kernel(
    _segmax_body,
    out_type=jax.ShapeDtypeStruct((NPAD, D), jnp.float32),
    mesh=plsc.VectorSubcoreMesh(core_axis_name="c", subcore_axis_name="s"),
    compiler_params=pltpu.CompilerParams(needs_layout_passes=False),
    scratch_types=[
        pltpu.VMEM((NPT + 1, D), jnp.float32),
        pltpu.VMEM((TABC,), jnp.int32),
        pltpu.VMEM((BLK,), jnp.int32),
        pltpu.VMEM((BLK,), jnp.int32),
        pltpu.VMEM((GRP,), jnp.int32),
        pltpu.VMEM((GRP, D), jnp.float32),
        pltpu.SemaphoreType.DMA,
    ],
)


def kernel(x, x_global, edge_attr, edge_index, batch_ind, num_graphs,
           f_mess_W, f_mess_b, f_agg_W, f_agg_b, att_mask_W, att_mask_b,
           att_feat_W, att_feat_b, transform_W, transform_b):
    del edge_attr, num_graphs
    src = edge_index[0]
    dst = edge_index[1]
    oh = (batch_ind[:, None] == jnp.arange(NG, dtype=batch_ind.dtype)[None, :]
          ).astype(jnp.float32)

    srcp, dstp, tab = _part_call(src, dst)
    y = _mess_call(x, f_mess_W[0], f_mess_b[0].reshape(1, D))
    for i in range(STEPS):
        araw = _segmax_call(y, srcp, dstp, tab)[:N]
        nxt = (i + 1) % STEPS
        x, x_global, y = _step_call(
            x, x_global, araw, oh,
            f_agg_W[i], f_agg_b[i].reshape(1, D),
            jnp.tile(att_mask_W[i], (NG, 1)), att_mask_b[i].reshape(1, 1),
            att_feat_W[i], att_feat_b[i].reshape(1, D),
            transform_W[i], transform_b[i].reshape(1, D),
            f_mess_W[nxt], f_mess_b[nxt].reshape(1, D),
        )
    return (x, x_global)
